# baseline (speedup 1.0000x reference)
import functools

import jax
import jax.numpy as jnp
from jax import lax
from jax.experimental import pallas as pl
from jax.experimental.pallas import tpu as pltpu
from jax.experimental.pallas import tpu_sc as plsc

BS, T, PN = 64, 1024, 32
EMB = 32
VOCAB = 21
NC, NS = 2, 16
NW = NC * NS
B_W = BS // NW
RT = 64
CHUNK = RT * PN
NCHUNK_B = T // RT
NCHUNK = B_W * NCHUNK_B

_mesh = plsc.VectorSubcoreMesh(
    core_axis_name="c", subcore_axis_name="s", num_cores=NC, num_subcores=NS
)


@functools.partial(
    pl.kernel,
    out_type=jax.ShapeDtypeStruct((BS, T, PN * EMB), jnp.float32),
    mesh=_mesh,
    scratch_types=[
        pltpu.VMEM_SHARED((VOCAB, EMB), jnp.float32),
        pltpu.VMEM((1, RT, PN, 1), jnp.int32),
        pltpu.VMEM((1, RT, PN * EMB), jnp.float32),
        pltpu.SemaphoreType.DMA,
    ],
    compiler_params=pltpu.CompilerParams(use_tc_tiling_on_sc=False),
)
def _patchify(spikes_hbm, table_hbm, out_hbm, table_sh, idx_v, rows_v, sem):
    cid = lax.axis_index("c")
    sid = lax.axis_index("s")
    wid = sid * NC + cid

    @pl.when(sid == 0)
    def _stage_table():
        pltpu.sync_copy(table_hbm, table_sh)

    plsc.subcore_barrier()

    def body(g, carry):
        b = B_W * wid + g // NCHUNK_B
        t0 = (g % NCHUNK_B) * RT
        pltpu.sync_copy(spikes_hbm.at[pl.ds(b, 1), pl.ds(t0, RT)], idx_v)
        pltpu.sync_copy(rows_v, out_hbm.at[pl.ds(b, 1), pl.ds(t0, RT)])
        return carry

    lax.fori_loop(0, NCHUNK, body, 0)


def kernel(spikes, table):
    return _patchify(spikes, table)


# speedup vs baseline: 8.4567x; 8.4567x over previous
import functools

import jax
import jax.numpy as jnp
from jax import lax
from jax.experimental import pallas as pl
from jax.experimental.pallas import tpu as pltpu
from jax.experimental.pallas import tpu_sc as plsc

BS, T, PN = 64, 1024, 32
EMB = 32
VOCAB = 21
N = BS * T * PN
NC, NS = 2, 16
NW = NC * NS
N_W = N // NW
CHUNK = 1024
NCHUNK = N_W // CHUNK
NPAIR = NCHUNK // 2

_mesh = plsc.VectorSubcoreMesh(
    core_axis_name="c", subcore_axis_name="s", num_cores=NC, num_subcores=NS
)


@functools.partial(
    pl.kernel,
    out_type=jax.ShapeDtypeStruct((N, EMB), jnp.float32),
    mesh=_mesh,
    scratch_types=[
        pltpu.VMEM_SHARED((VOCAB, EMB), jnp.float32),
        pltpu.VMEM((2, CHUNK), jnp.int32),
        pltpu.VMEM((2 * CHUNK, EMB), jnp.float32),
        pltpu.SemaphoreType.DMA,
        pltpu.SemaphoreType.DMA,
        pltpu.SemaphoreType.DMA,
        pltpu.SemaphoreType.DMA,
        pltpu.SemaphoreType.DMA,
    ],
    compiler_params=pltpu.CompilerParams(use_tc_tiling_on_sc=False),
)
def _patchify(
    idx_hbm, table_hbm, out_hbm, table_sh, idx_v, rows_v,
    si0, si1, sg, so0, so1,
):
    cid = lax.axis_index("c")
    sid = lax.axis_index("s")
    wid = sid * NC + cid
    base = wid * N_W

    @pl.when(sid == 0)
    def _stage_table():
        pltpu.sync_copy(table_hbm, table_sh)

    plsc.subcore_barrier()

    def idx_load(g, buf, sem):
        return pltpu.make_async_copy(
            idx_hbm.at[pl.ds(wid, 1), pl.ds(g * CHUNK, CHUNK)],
            idx_v.at[pl.ds(buf, 1)],
            sem,
        )

    def gather(buf):
        return pltpu.make_async_copy(
            table_sh.at[idx_v.at[pl.ds(buf, 1)].at[0]],
            rows_v.at[pl.ds(buf * CHUNK, CHUNK)],
            sg,
        )

    def store(g, buf, sem):
        return pltpu.make_async_copy(
            rows_v.at[pl.ds(buf * CHUNK, CHUNK)],
            out_hbm.at[pl.ds(base + g * CHUNK, CHUNK)],
            sem,
        )

    idx_load(0, 0, si0).start()
    idx_load(1, 1, si1).start()

    def half(gp, buf, sem_i, sem_o):
        g = 2 * gp + buf
        idx_load(g, buf, sem_i).wait()

        @pl.when(g >= 2)
        def _drain_store():
            store(g - 2, buf, sem_o).wait()

        ga = gather(buf)
        ga.start()
        ga.wait()
        store(g, buf, sem_o).start()

        @pl.when(g + 2 < NCHUNK)
        def _next_idx():
            idx_load(g + 2, buf, sem_i).start()

    def body(gp, carry):
        half(gp, 0, si0, so0)
        half(gp, 1, si1, so1)
        return carry

    lax.fori_loop(0, NPAIR, body, 0)
    store(NCHUNK - 2, 0, so0).wait()
    store(NCHUNK - 1, 1, so1).wait()


def kernel(spikes, table):
    idx = spikes.reshape(NW, N_W)
    out = _patchify(idx, table)
    return out.reshape(BS, T, PN * EMB)
